# baseline (speedup 1.0000x reference)
import functools

import jax
import jax.numpy as jnp
from jax import lax
from jax.experimental import pallas as pl
from jax.experimental.pallas import tpu as pltpu
from jax.experimental.pallas import tpu_sc as plsc

N = 10000
E = 320000
IN_DIM = 128
D = 64
NC = 2
NS = 16
NW = NC * NS
B = 80
NB = 125
NPAD = 10240
RPS = NPAD // NS

_MESH = plsc.VectorSubcoreMesh(core_axis_name="c", subcore_axis_name="s")


@functools.partial(
    pl.kernel,
    out_type=jax.ShapeDtypeStruct((NC, NPAD), jnp.float32),
    mesh=_MESH,
    scratch_types=[
        pltpu.VMEM_SHARED((NPAD,), jnp.float32),
        pltpu.VMEM((NB, B), jnp.int32),
        pltpu.VMEM((B,), jnp.float32),
        pltpu.VMEM((RPS,), jnp.float32),
    ],
)
def _deg(dst_hbm, out_hbm, acc, didx, ones_v, zbuf):
    c = lax.axis_index("c")
    s = lax.axis_index("s")
    wid = s * NC + c

    def zfill(i, carry):
        zbuf[pl.ds(i * 16, 16)] = jnp.zeros((16,), jnp.float32)
        return carry

    lax.fori_loop(0, RPS // 16, zfill, 0)
    pltpu.sync_copy(zbuf, acc.at[pl.ds(s * RPS, RPS)])
    for i in range(B // 16):
        ones_v[pl.ds(i * 16, 16)] = jnp.ones((16,), jnp.float32)
    pltpu.sync_copy(dst_hbm.at[wid], didx)
    plsc.subcore_barrier()

    def body(j, carry):
        pltpu.sync_copy(ones_v, acc.at[didx.at[j]], add=True)
        return carry

    lax.fori_loop(0, NB, body, 0)
    plsc.subcore_barrier()
    pltpu.sync_copy(acc.at[pl.ds(s * RPS, RPS)], out_hbm.at[c, pl.ds(s * RPS, RPS)])


@functools.partial(
    pl.kernel,
    out_type=jax.ShapeDtypeStruct((NC, NPAD, D), jnp.float32),
    mesh=_MESH,
    scratch_types=[
        pltpu.VMEM_SHARED((NPAD, D), jnp.float32),
        pltpu.VMEM_SHARED((N, D), jnp.float32),
        pltpu.VMEM((NB, B), jnp.int32),
        pltpu.VMEM((NB, B), jnp.int32),
        [pltpu.VMEM((B, D), jnp.float32)] * 2,
        [pltpu.SemaphoreType.DMA] * 2,
        [pltpu.SemaphoreType.DMA] * 2,
    ],
    compiler_params=pltpu.CompilerParams(use_tc_tiling_on_sc=False),
)
def _prop(tbl_hbm, src_hbm, dst_hbm, out_hbm, acc, tbl_sh, sidx, didx,
          rows, gsem, ssem):
    c = lax.axis_index("c")
    s = lax.axis_index("s")
    wid = s * NC + c
    pltpu.async_copy(src_hbm.at[wid], sidx, gsem[0])
    pltpu.async_copy(dst_hbm.at[wid], didx, gsem[1])
    pltpu.async_copy(tbl_hbm.at[pl.ds(s * (N // NS), N // NS)],
                     tbl_sh.at[pl.ds(s * (N // NS), N // NS)], ssem[0])

    def zfill(r, carry):
        for k in range(D // 16):
            rows[0][r, pl.ds(k * 16, 16)] = jnp.zeros((16,), jnp.float32)
        return carry

    lax.fori_loop(0, B, zfill, 0)
    for k in range(RPS // B):
        pltpu.sync_copy(rows[0], acc.at[pl.ds(s * RPS + k * B, B)])
    pltpu.make_async_copy(src_hbm.at[wid], sidx, gsem[0]).wait()
    pltpu.make_async_copy(dst_hbm.at[wid], didx, gsem[1]).wait()
    pltpu.make_async_copy(tbl_hbm.at[pl.ds(s * (N // NS), N // NS)],
                          tbl_sh.at[pl.ds(s * (N // NS), N // NS)], ssem[0]).wait()
    plsc.subcore_barrier()
    for b in range(2):
        pltpu.async_copy(tbl_sh.at[sidx.at[b]], rows[b], gsem[b])

    def body(jj, carry):
        j0 = jj * 2
        for b in range(2):
            pltpu.make_async_copy(tbl_sh.at[sidx.at[j0 + b]], rows[b], gsem[b]).wait()
            pltpu.async_copy(rows[b], acc.at[didx.at[j0 + b]], ssem[b], add=True)
        pltpu.make_async_copy(rows[0], acc.at[didx.at[j0]], ssem[0]).wait()
        pltpu.async_copy(tbl_sh.at[sidx.at[j0 + 2]], rows[0], gsem[0])
        pltpu.make_async_copy(rows[1], acc.at[didx.at[j0 + 1]], ssem[1]).wait()

        @pl.when(jj < NB // 2 - 1)
        def _():
            pltpu.async_copy(tbl_sh.at[sidx.at[j0 + 3]], rows[1], gsem[1])
        return carry

    lax.fori_loop(0, NB // 2, body, 0)
    pltpu.make_async_copy(tbl_sh.at[sidx.at[NB - 1]], rows[0], gsem[0]).wait()
    pltpu.sync_copy(rows[0], acc.at[didx.at[NB - 1]], add=True)
    plsc.subcore_barrier()
    pltpu.sync_copy(acc.at[pl.ds(s * RPS, RPS)], out_hbm.at[c, pl.ds(s * RPS, RPS)])


def _tc1_body(x_ref, w_ref, o_ref):
    o_ref[...] = jnp.dot(x_ref[...], w_ref[...], preferred_element_type=jnp.float32)


def _tc_mid_body(ps_ref, h_ref, dinv_ref, b_ref, w_ref, o_ref):
    z = jnp.maximum((ps_ref[...] + h_ref[...] * dinv_ref[...]) * dinv_ref[...] + b_ref[...], 0.0)
    o_ref[...] = jnp.dot(z, w_ref[...], preferred_element_type=jnp.float32)


def _tc_fin_body(ps_ref, h_ref, dinv_ref, b_ref, w_ref, blin_ref, o_ref):
    z = jnp.maximum((ps_ref[...] + h_ref[...] * dinv_ref[...]) * dinv_ref[...] + b_ref[...], 0.0)
    o_ref[...] = jnp.dot(z, w_ref[...], preferred_element_type=jnp.float32) + blin_ref[...]


def kernel(x, edge_index, W1, b1, W2, b2, lin_W, b_lin):
    f32 = jnp.float32
    ei = edge_index.astype(jnp.int32)
    srcp = ei[0].reshape(NW, NB, B)
    dstp = ei[1].reshape(NW, NB, B)
    degp = _deg(dstp)
    deg = degp[0, :N] + degp[1, :N] + 1.0
    dinv = lax.rsqrt(deg)[:, None]

    h1 = pl.pallas_call(
        _tc1_body, out_shape=jax.ShapeDtypeStruct((N, D), f32),
    )(x, W1)
    h1p = h1 * dinv

    p1 = _prop(h1p, srcp, dstp)
    ps1 = p1[0, :N] + p1[1, :N]

    h2 = pl.pallas_call(
        _tc_mid_body, out_shape=jax.ShapeDtypeStruct((N, D), f32),
    )(ps1, h1, dinv, b1.reshape(1, D), W2)
    h2p = h2 * dinv

    p2 = _prop(h2p, srcp, dstp)
    ps2 = p2[0, :N] + p2[1, :N]

    logits = pl.pallas_call(
        _tc_fin_body, out_shape=jax.ShapeDtypeStruct((N, 2), f32),
    )(ps2, h2, dinv, b2.reshape(1, D), lin_W, b_lin.reshape(1, 2))
    return logits


# speedup vs baseline: 1.0240x; 1.0240x over previous
import functools

import jax
import jax.numpy as jnp
from jax import lax
from jax.experimental import pallas as pl
from jax.experimental.pallas import tpu as pltpu
from jax.experimental.pallas import tpu_sc as plsc

N = 10000
E = 320000
IN_DIM = 128
D = 64
NC = 2
NS = 16
NW = NC * NS
B = 80
NB = 125
NPAD = 10240
RPS = NPAD // NS

_MESH = plsc.VectorSubcoreMesh(core_axis_name="c", subcore_axis_name="s")


@functools.partial(
    pl.kernel,
    out_type=jax.ShapeDtypeStruct((NC, NPAD), jnp.float32),
    mesh=_MESH,
    scratch_types=[
        pltpu.VMEM_SHARED((NPAD,), jnp.float32),
        pltpu.VMEM((NB, B), jnp.int32),
        pltpu.VMEM((B,), jnp.float32),
        pltpu.VMEM((RPS,), jnp.float32),
    ],
)
def _deg(dst_hbm, out_hbm, acc, didx, ones_v, zbuf):
    c = lax.axis_index("c")
    s = lax.axis_index("s")
    wid = s * NC + c

    def zfill(i, carry):
        zbuf[pl.ds(i * 16, 16)] = jnp.zeros((16,), jnp.float32)
        return carry

    lax.fori_loop(0, RPS // 16, zfill, 0)
    pltpu.sync_copy(zbuf, acc.at[pl.ds(s * RPS, RPS)])
    for i in range(B // 16):
        ones_v[pl.ds(i * 16, 16)] = jnp.ones((16,), jnp.float32)
    pltpu.sync_copy(dst_hbm.at[wid], didx)
    plsc.subcore_barrier()

    def body(j, carry):
        pltpu.sync_copy(ones_v, acc.at[didx.at[j]], add=True)
        return carry

    lax.fori_loop(0, NB, body, 0)
    plsc.subcore_barrier()
    pltpu.sync_copy(acc.at[pl.ds(s * RPS, RPS)], out_hbm.at[c, pl.ds(s * RPS, RPS)])


@functools.partial(
    pl.kernel,
    out_type=jax.ShapeDtypeStruct((NC, NPAD, D), jnp.float32),
    mesh=_MESH,
    scratch_types=[
        pltpu.VMEM_SHARED((NPAD, D), jnp.float32),
        pltpu.VMEM_SHARED((N, D), jnp.float32),
        pltpu.VMEM((NB, B), jnp.int32),
        pltpu.VMEM((NB, B), jnp.int32),
        [pltpu.VMEM((B, D), jnp.float32)] * 2,
        [pltpu.SemaphoreType.DMA] * 2,
        [pltpu.SemaphoreType.DMA] * 2,
    ],
    compiler_params=pltpu.CompilerParams(use_tc_tiling_on_sc=False),
)
def _prop(tbl_hbm, src_hbm, dst_hbm, out_hbm, acc, tbl_sh, sidx, didx,
          rows, gsem, ssem):
    c = lax.axis_index("c")
    s = lax.axis_index("s")
    wid = s * NC + c
    pltpu.async_copy(src_hbm.at[wid], sidx, gsem[0])
    pltpu.async_copy(dst_hbm.at[wid], didx, gsem[1])
    pltpu.async_copy(tbl_hbm.at[pl.ds(s * (N // NS), N // NS)],
                     tbl_sh.at[pl.ds(s * (N // NS), N // NS)], ssem[0])

    def zfill(r, carry):
        for k in range(D // 16):
            rows[0][r, pl.ds(k * 16, 16)] = jnp.zeros((16,), jnp.float32)
        return carry

    lax.fori_loop(0, B, zfill, 0)
    for k in range(RPS // B):
        pltpu.sync_copy(rows[0], acc.at[pl.ds(s * RPS + k * B, B)])
    pltpu.make_async_copy(src_hbm.at[wid], sidx, gsem[0]).wait()
    pltpu.make_async_copy(dst_hbm.at[wid], didx, gsem[1]).wait()
    pltpu.make_async_copy(tbl_hbm.at[pl.ds(s * (N // NS), N // NS)],
                          tbl_sh.at[pl.ds(s * (N // NS), N // NS)], ssem[0]).wait()
    plsc.subcore_barrier()
    for b in range(2):
        pltpu.async_copy(tbl_sh.at[sidx.at[b]], rows[b], gsem[b])

    def body(jj, carry):
        j0 = jj * 2
        for b in range(2):
            pltpu.make_async_copy(tbl_sh.at[sidx.at[j0 + b]], rows[b], gsem[b]).wait()
            pltpu.async_copy(rows[b], acc.at[didx.at[j0 + b]], ssem[b], add=True)
        pltpu.make_async_copy(rows[0], acc.at[didx.at[j0]], ssem[0]).wait()
        pltpu.async_copy(tbl_sh.at[sidx.at[j0 + 2]], rows[0], gsem[0])
        pltpu.make_async_copy(rows[1], acc.at[didx.at[j0 + 1]], ssem[1]).wait()

        @pl.when(jj < NB // 2 - 1)
        def _():
            pltpu.async_copy(tbl_sh.at[sidx.at[j0 + 3]], rows[1], gsem[1])
        return carry

    lax.fori_loop(0, NB // 2, body, 0)
    pltpu.make_async_copy(tbl_sh.at[sidx.at[NB - 1]], rows[0], gsem[0]).wait()
    pltpu.sync_copy(rows[0], acc.at[didx.at[NB - 1]], add=True)
    plsc.subcore_barrier()
    pltpu.sync_copy(acc.at[pl.ds(s * RPS, RPS)], out_hbm.at[c, pl.ds(s * RPS, RPS)])


def _tc1_body(x_ref, w_ref, o_ref):
    o_ref[...] = jnp.dot(x_ref[...], w_ref[...], preferred_element_type=jnp.float32)


def _tc_mid_body(p_ref, h_ref, dinv_ref, b_ref, w_ref, o_ref):
    ps = p_ref[0, :N, :] + p_ref[1, :N, :]
    z = jnp.maximum((ps + h_ref[...] * dinv_ref[...]) * dinv_ref[...] + b_ref[...], 0.0)
    o_ref[...] = jnp.dot(z, w_ref[...], preferred_element_type=jnp.float32)


def _tc_fin_body(p_ref, h_ref, dinv_ref, b_ref, w_ref, blin_ref, o_ref):
    ps = p_ref[0, :N, :] + p_ref[1, :N, :]
    z = jnp.maximum((ps + h_ref[...] * dinv_ref[...]) * dinv_ref[...] + b_ref[...], 0.0)
    o_ref[...] = jnp.dot(z, w_ref[...], preferred_element_type=jnp.float32) + blin_ref[...]


def kernel(x, edge_index, W1, b1, W2, b2, lin_W, b_lin):
    f32 = jnp.float32
    ei = edge_index.astype(jnp.int32)
    srcp = ei[0].reshape(NW, NB, B)
    dstp = ei[1].reshape(NW, NB, B)
    degp = _deg(dstp)
    deg = degp[0, :N] + degp[1, :N] + 1.0
    dinv = lax.rsqrt(deg)[:, None]

    h1 = pl.pallas_call(
        _tc1_body, out_shape=jax.ShapeDtypeStruct((N, D), f32),
    )(x, W1)
    h1p = h1 * dinv

    p1 = _prop(h1p, srcp, dstp)

    h2 = pl.pallas_call(
        _tc_mid_body, out_shape=jax.ShapeDtypeStruct((N, D), f32),
    )(p1, h1, dinv, b1.reshape(1, D), W2)
    h2p = h2 * dinv

    p2 = _prop(h2p, srcp, dstp)

    logits = pl.pallas_call(
        _tc_fin_body, out_shape=jax.ShapeDtypeStruct((N, 2), f32),
    )(p2, h2, dinv, b2.reshape(1, D), lin_W, b_lin.reshape(1, 2))
    return logits


# speedup vs baseline: 1.1003x; 1.0745x over previous
import functools

import jax
import jax.numpy as jnp
from jax import lax
from jax.experimental import pallas as pl
from jax.experimental.pallas import tpu as pltpu
from jax.experimental.pallas import tpu_sc as plsc

N = 10000
E = 320000
IN_DIM = 128
D = 64
NC = 2
NS = 16
NW = NC * NS
B = 128
NBLK = E // B
NBW = 79
NBQ = 39
NPAD = 10240
RPS = NPAD // NS

_MESH = plsc.VectorSubcoreMesh(core_axis_name="c", subcore_axis_name="s")


@functools.partial(
    pl.kernel,
    out_type=jax.ShapeDtypeStruct((NC, NPAD), jnp.float32),
    mesh=_MESH,
    scratch_types=[
        pltpu.VMEM_SHARED((NPAD,), jnp.float32),
        pltpu.VMEM((NBW, 2, B), jnp.int32),
        pltpu.VMEM((B,), jnp.float32),
        pltpu.VMEM((RPS,), jnp.float32),
    ],
)
def _deg(ei_hbm, out_hbm, acc, eidx, ones_v, zbuf):
    c = lax.axis_index("c")
    s = lax.axis_index("s")
    wid = s * NC + c
    lo = jnp.where(wid < 4, wid * 79, wid * 78 + 4)
    trip = 78 + jnp.where(wid < 4, 1, 0)
    pltpu.sync_copy(ei_hbm.at[pl.ds(lo, 78)], eidx.at[pl.ds(0, 78)])

    @pl.when(wid < 4)
    def _():
        pltpu.sync_copy(ei_hbm.at[pl.ds(lo + 78, 1)], eidx.at[pl.ds(78, 1)])

    def zfill(i, carry):
        zbuf[pl.ds(i * 16, 16)] = jnp.zeros((16,), jnp.float32)
        return carry

    lax.fori_loop(0, RPS // 16, zfill, 0)
    pltpu.sync_copy(zbuf, acc.at[pl.ds(s * RPS, RPS)])
    for i in range(B // 16):
        ones_v[pl.ds(i * 16, 16)] = jnp.ones((16,), jnp.float32)
    plsc.subcore_barrier()

    def body(j, carry):
        pltpu.sync_copy(ones_v, acc.at[eidx.at[j, 1]], add=True)
        return carry

    lax.fori_loop(0, trip, body, 0)
    plsc.subcore_barrier()
    pltpu.sync_copy(acc.at[pl.ds(s * RPS, RPS)], out_hbm.at[c, pl.ds(s * RPS, RPS)])


@functools.partial(
    pl.kernel,
    out_type=jax.ShapeDtypeStruct((NC, NPAD, D), jnp.float32),
    mesh=_MESH,
    scratch_types=[
        pltpu.VMEM_SHARED((NPAD, D), jnp.float32),
        pltpu.VMEM_SHARED((N, D), jnp.float32),
        pltpu.VMEM((NBW, 2, B), jnp.int32),
        [pltpu.VMEM((B, D), jnp.float32)] * 2,
        [pltpu.SemaphoreType.DMA] * 2,
        [pltpu.SemaphoreType.DMA] * 2,
    ],
    compiler_params=pltpu.CompilerParams(use_tc_tiling_on_sc=False),
)
def _prop(tbl_hbm, ei_hbm, out_hbm, acc, tbl_sh, eidx,
          rows, gsem, ssem):
    c = lax.axis_index("c")
    s = lax.axis_index("s")
    wid = s * NC + c
    lo = jnp.where(wid < 4, wid * 79, wid * 78 + 4)
    trip = 78 + jnp.where(wid < 4, 1, 0)
    pltpu.async_copy(ei_hbm.at[pl.ds(lo, 78)], eidx.at[pl.ds(0, 78)], gsem[0])

    @pl.when(wid < 4)
    def _():
        pltpu.async_copy(ei_hbm.at[pl.ds(lo + 78, 1)], eidx.at[pl.ds(78, 1)], gsem[1])

    pltpu.async_copy(tbl_hbm.at[pl.ds(s * (N // NS), N // NS)],
                     tbl_sh.at[pl.ds(s * (N // NS), N // NS)], ssem[0])

    def zfill(r, carry):
        for k in range(D // 16):
            rows[0][r, pl.ds(k * 16, 16)] = jnp.zeros((16,), jnp.float32)
        return carry

    lax.fori_loop(0, B, zfill, 0)
    for k in range(RPS // B):
        pltpu.sync_copy(rows[0], acc.at[pl.ds(s * RPS + k * B, B)])
    pltpu.make_async_copy(ei_hbm.at[pl.ds(lo, 78)], eidx.at[pl.ds(0, 78)], gsem[0]).wait()

    @pl.when(wid < 4)
    def _():
        pltpu.make_async_copy(ei_hbm.at[pl.ds(lo + 78, 1)],
                              eidx.at[pl.ds(78, 1)], gsem[1]).wait()

    pltpu.make_async_copy(tbl_hbm.at[pl.ds(s * (N // NS), N // NS)],
                          tbl_sh.at[pl.ds(s * (N // NS), N // NS)], ssem[0]).wait()
    plsc.subcore_barrier()
    for b in range(2):
        pltpu.async_copy(tbl_sh.at[eidx.at[b, 0]], rows[b], gsem[b])

    def body(jj, carry):
        j0 = jj * 2
        for b in range(2):
            pltpu.make_async_copy(tbl_sh.at[eidx.at[j0 + b, 0]], rows[b], gsem[b]).wait()
            pltpu.async_copy(rows[b], acc.at[eidx.at[j0 + b, 1]], ssem[b], add=True)
        pltpu.make_async_copy(rows[0], acc.at[eidx.at[j0, 1]], ssem[0]).wait()

        @pl.when(j0 + 2 < trip)
        def _():
            pltpu.async_copy(tbl_sh.at[eidx.at[j0 + 2, 0]], rows[0], gsem[0])

        pltpu.make_async_copy(rows[1], acc.at[eidx.at[j0 + 1, 1]], ssem[1]).wait()

        @pl.when(j0 + 3 < trip)
        def _():
            pltpu.async_copy(tbl_sh.at[eidx.at[j0 + 3, 0]], rows[1], gsem[1])
        return carry

    lax.fori_loop(0, NBQ, body, 0)

    @pl.when(trip > 78)
    def _():
        pltpu.make_async_copy(tbl_sh.at[eidx.at[78, 0]], rows[0], gsem[0]).wait()
        pltpu.sync_copy(rows[0], acc.at[eidx.at[78, 1]], add=True)
    plsc.subcore_barrier()
    pltpu.sync_copy(acc.at[pl.ds(s * RPS, RPS)], out_hbm.at[c, pl.ds(s * RPS, RPS)])


def _tc1_body(x_ref, w_ref, o_ref):
    o_ref[...] = jnp.dot(x_ref[...], w_ref[...], preferred_element_type=jnp.float32)


def _tc_mid_body(p_ref, h_ref, dinv_ref, b_ref, w_ref, o_ref):
    ps = p_ref[0, :N, :] + p_ref[1, :N, :]
    z = jnp.maximum((ps + h_ref[...] * dinv_ref[...]) * dinv_ref[...] + b_ref[...], 0.0)
    o_ref[...] = jnp.dot(z, w_ref[...], preferred_element_type=jnp.float32)


def _tc_fin_body(p_ref, h_ref, dinv_ref, b_ref, w_ref, blin_ref, o_ref):
    ps = p_ref[0, :N, :] + p_ref[1, :N, :]
    z = jnp.maximum((ps + h_ref[...] * dinv_ref[...]) * dinv_ref[...] + b_ref[...], 0.0)
    o_ref[...] = jnp.dot(z, w_ref[...], preferred_element_type=jnp.float32) + blin_ref[...]


def kernel(x, edge_index, W1, b1, W2, b2, lin_W, b_lin):
    f32 = jnp.float32
    ei3 = jnp.transpose(edge_index.astype(jnp.int32).reshape(2, NBLK, B), (1, 0, 2))
    degp = _deg(ei3)
    deg = degp[0, :N] + degp[1, :N] + 1.0
    dinv = lax.rsqrt(deg)[:, None]

    h1 = pl.pallas_call(
        _tc1_body, out_shape=jax.ShapeDtypeStruct((N, D), f32),
    )(x, W1)
    h1p = h1 * dinv

    p1 = _prop(h1p, ei3)

    h2 = pl.pallas_call(
        _tc_mid_body, out_shape=jax.ShapeDtypeStruct((N, D), f32),
    )(p1, h1, dinv, b1.reshape(1, D), W2)
    h2p = h2 * dinv

    p2 = _prop(h2p, ei3)

    logits = pl.pallas_call(
        _tc_fin_body, out_shape=jax.ShapeDtypeStruct((N, 2), f32),
    )(p2, h2, dinv, b2.reshape(1, D), lin_W, b_lin.reshape(1, 2))
    return logits
